# baseline (speedup 1.0000x reference)
import jax
import jax.numpy as jnp
from jax import lax
from jax.experimental import pallas as pl
from jax.experimental.pallas import tpu as pltpu
from jax.experimental.pallas import tpu_sc as plsc

B = 16384
D = 64
H = 32
W = 128
PK = W // H
NC = 2
NS = 16
NW = NC * NS
BPW = B // NW
L = 16
CH = 64
NCH = BPW // CH
NG = CH // L


def _load_gather(ref, indices):
    return plsc.load_gather(ref, indices)


def _fm_body(u_hbm, i_hbm, j_hbm, euA, euB, eiA, eiB, ub_hbm, ib_hbm, b_hbm,
             out_i, out_j,
             uidx, iidx, jidx, pidx,
             urA, urB, irA, irB, jrA, jrB,
             ubl, ibl, jbl, bv, res_i, res_j, sem):
    wid = lax.axis_index("c") * NS + lax.axis_index("s")
    base = wid * BPW

    pltpu.sync_copy(u_hbm.at[pl.ds(base, BPW)], uidx)
    pltpu.sync_copy(i_hbm.at[pl.ds(base, BPW)], iidx)
    pltpu.sync_copy(j_hbm.at[pl.ds(base, BPW)], jidx)
    pltpu.sync_copy(b_hbm, bv)

    iota16 = lax.iota(jnp.int32, L)

    def pbody(q, _):
        s = pl.ds(q * L, L)
        pidx[0, s] = uidx[s] >> 2
        pidx[1, s] = iidx[s] >> 2
        pidx[2, s] = jidx[s] >> 2
        pidx[3, s] = uidx[s] >> 7
        pidx[4, s] = iidx[s] >> 7
        pidx[5, s] = jidx[s] >> 7
        return 0

    lax.fori_loop(0, BPW // L, pbody, 0)

    def chunk(k, _):
        cs = pl.ds(k * CH, CH)
        cps = [pltpu.async_copy(euA.at[pidx.at[0, cs]], urA, sem),
               pltpu.async_copy(euB.at[pidx.at[0, cs]], urB, sem),
               pltpu.async_copy(eiA.at[pidx.at[1, cs]], irA, sem),
               pltpu.async_copy(eiB.at[pidx.at[1, cs]], irB, sem),
               pltpu.async_copy(eiA.at[pidx.at[2, cs]], jrA, sem),
               pltpu.async_copy(eiB.at[pidx.at[2, cs]], jrB, sem),
               pltpu.async_copy(ub_hbm.at[pidx.at[3, cs]], ubl, sem),
               pltpu.async_copy(ib_hbm.at[pidx.at[4, cs]], ibl, sem),
               pltpu.async_copy(ib_hbm.at[pidx.at[5, cs]], jbl, sem)]
        for cp in cps:
            cp.wait()
        bias = bv[...]

        def gbody(g, _):
            s = pl.ds(k * CH + g * L, L)
            uoff = (uidx[s] & (PK - 1)) * H
            ioff = (iidx[s] & (PK - 1)) * H
            joff = (jidx[s] & (PK - 1)) * H
            ids = g * L + iota16

            def fbody(f, carry):
                acc_i, acc_j = carry
                ua = _load_gather(urA, [ids, uoff + f])
                ia = _load_gather(irA, [ids, ioff + f])
                ja = _load_gather(jrA, [ids, joff + f])
                ub_ = _load_gather(urB, [ids, uoff + f])
                ib_ = _load_gather(irB, [ids, ioff + f])
                jb_ = _load_gather(jrB, [ids, joff + f])
                return (acc_i + ua * ia + ub_ * ib_,
                        acc_j + ua * ja + ub_ * jb_)

            bu = _load_gather(ubl, [ids, uidx[s] & (W - 1)])
            bi = _load_gather(ibl, [ids, iidx[s] & (W - 1)])
            bj = _load_gather(jbl, [ids, jidx[s] & (W - 1)])
            acc0 = jnp.zeros((L,), jnp.float32)
            acc_i, acc_j = lax.fori_loop(0, H, fbody, (acc0, acc0), unroll=8)
            res_i[s] = acc_i + bu + bi + bias
            res_j[s] = acc_j + bu + bj + bias
            return 0

        lax.fori_loop(0, NG, gbody, 0)
        return 0

    lax.fori_loop(0, NCH, chunk, 0)

    pltpu.sync_copy(res_i, out_i.at[pl.ds(base, BPW)])
    pltpu.sync_copy(res_j, out_j.at[pl.ds(base, BPW)])


@jax.jit
def _pair_fm(u1, i1, j1, euA, euB, eiA, eiB, ub2, ib2, b16):
    mesh = plsc.VectorSubcoreMesh(core_axis_name="c", subcore_axis_name="s",
                                  num_cores=NC, num_subcores=NS)
    f = pl.kernel(
        _fm_body,
        out_type=[jax.ShapeDtypeStruct((B,), jnp.float32),
                  jax.ShapeDtypeStruct((B,), jnp.float32)],
        mesh=mesh,
        compiler_params=pltpu.CompilerParams(needs_layout_passes=False,
                                             use_tc_tiling_on_sc=True),
        scratch_types=[
            pltpu.VMEM((BPW,), jnp.int32),
            pltpu.VMEM((BPW,), jnp.int32),
            pltpu.VMEM((BPW,), jnp.int32),
            pltpu.VMEM((6, BPW), jnp.int32),
            pltpu.VMEM((CH, W), jnp.float32),
            pltpu.VMEM((CH, W), jnp.float32),
            pltpu.VMEM((CH, W), jnp.float32),
            pltpu.VMEM((CH, W), jnp.float32),
            pltpu.VMEM((CH, W), jnp.float32),
            pltpu.VMEM((CH, W), jnp.float32),
            pltpu.VMEM((CH, W), jnp.float32),
            pltpu.VMEM((CH, W), jnp.float32),
            pltpu.VMEM((CH, W), jnp.float32),
            pltpu.VMEM((L,), jnp.float32),
            pltpu.VMEM((BPW,), jnp.float32),
            pltpu.VMEM((BPW,), jnp.float32),
            pltpu.SemaphoreType.DMA,
        ],
    )
    return f(u1, i1, j1, euA, euB, eiA, eiB, ub2, ib2, b16)


def kernel(u, i, j, c, embed_user, embed_item, u_bias, i_bias, bias_):
    del c
    u1 = u.astype(jnp.int32)
    i1 = i.astype(jnp.int32)
    j1 = j.astype(jnp.int32)
    euA = embed_user[:, :H].reshape(-1, W)
    euB = embed_user[:, H:].reshape(-1, W)
    eiA = embed_item[:, :H].reshape(-1, W)
    eiB = embed_item[:, H:].reshape(-1, W)
    nu = u_bias.shape[0]
    ni = i_bias.shape[0]
    ub2 = jnp.pad(u_bias.reshape(-1), (0, (-nu) % W)).reshape(-1, W)
    ib2 = jnp.pad(i_bias.reshape(-1), (0, (-ni) % W)).reshape(-1, W)
    b16 = jnp.broadcast_to(bias_, (L,))
    return tuple(_pair_fm(u1, i1, j1, euA, euB, eiA, eiB, ub2, ib2, b16))


# speedup vs baseline: 2.5136x; 2.5136x over previous
import jax
import jax.numpy as jnp
from jax import lax
from jax.experimental import pallas as pl
from jax.experimental.pallas import tpu as pltpu
from jax.experimental.pallas import tpu_sc as plsc

B = 16384
D = 64
W = 128
NC = 2
NS = 16
NW = NC * NS
BPW = B // NW
L = 16
CH = 128
NCH = BPW // CH
NG = CH // L


def _load_gather(ref, indices):
    return plsc.load_gather(ref, indices)


def _fm_body(u_hbm, i_hbm, j_hbm, eu_hbm, ei_hbm, ub_hbm, ib_hbm, b_hbm,
             out_i, out_j,
             uidx, iidx, jidx, pidx, urows, irows, jrows,
             ubl, ibl, jbl, bv, res_i, res_j, sem):
    wid = lax.axis_index("c") * NS + lax.axis_index("s")
    base = wid * BPW

    pltpu.sync_copy(u_hbm.at[pl.ds(base, BPW)], uidx)
    pltpu.sync_copy(i_hbm.at[pl.ds(base, BPW)], iidx)
    pltpu.sync_copy(j_hbm.at[pl.ds(base, BPW)], jidx)
    pltpu.sync_copy(b_hbm, bv)

    iota16 = lax.iota(jnp.int32, L)

    def pbody(q, _):
        s = pl.ds(q * L, L)
        pidx[0, s] = uidx[s]
        pidx[1, s] = iidx[s]
        pidx[2, s] = jidx[s]
        pidx[3, s] = uidx[s] >> 7
        pidx[4, s] = iidx[s] >> 7
        pidx[5, s] = jidx[s] >> 7
        return 0

    lax.fori_loop(0, BPW // L, pbody, 0)

    def chunk(k, _):
        cs = pl.ds(k * CH, CH)
        cps = [pltpu.async_copy(eu_hbm.at[pidx.at[0, cs]], urows, sem),
               pltpu.async_copy(ei_hbm.at[pidx.at[1, cs]], irows, sem),
               pltpu.async_copy(ei_hbm.at[pidx.at[2, cs]], jrows, sem),
               pltpu.async_copy(ub_hbm.at[pidx.at[3, cs]], ubl, sem),
               pltpu.async_copy(ib_hbm.at[pidx.at[4, cs]], ibl, sem),
               pltpu.async_copy(ib_hbm.at[pidx.at[5, cs]], jbl, sem)]
        for cp in cps:
            cp.wait()
        bias = bv[...]

        def gbody(g, _):
            s = pl.ds(k * CH + g * L, L)
            ids = g * L + iota16

            def fbody(f, carry):
                acc_i, acc_j = carry
                fv = ids * 0 + f
                ue = _load_gather(urows, [ids, fv])
                ie = _load_gather(irows, [ids, fv])
                je = _load_gather(jrows, [ids, fv])
                return acc_i + ue * ie, acc_j + ue * je

            bu = _load_gather(ubl, [ids, uidx[s] & (W - 1)])
            bi = _load_gather(ibl, [ids, iidx[s] & (W - 1)])
            bj = _load_gather(jbl, [ids, jidx[s] & (W - 1)])
            acc0 = jnp.zeros((L,), jnp.float32)
            acc_i, acc_j = lax.fori_loop(0, D, fbody, (acc0, acc0), unroll=8)
            res_i[s] = acc_i + bu + bi + bias
            res_j[s] = acc_j + bu + bj + bias
            return 0

        lax.fori_loop(0, NG, gbody, 0)
        return 0

    lax.fori_loop(0, NCH, chunk, 0)

    pltpu.sync_copy(res_i, out_i.at[pl.ds(base, BPW)])
    pltpu.sync_copy(res_j, out_j.at[pl.ds(base, BPW)])


@jax.jit
def _pair_fm(u1, i1, j1, eu2, ei2, ub2, ib2, b16):
    mesh = plsc.VectorSubcoreMesh(core_axis_name="c", subcore_axis_name="s",
                                  num_cores=NC, num_subcores=NS)
    f = pl.kernel(
        _fm_body,
        out_type=[jax.ShapeDtypeStruct((B,), jnp.float32),
                  jax.ShapeDtypeStruct((B,), jnp.float32)],
        mesh=mesh,
        compiler_params=pltpu.CompilerParams(needs_layout_passes=False,
                                             use_tc_tiling_on_sc=True),
        scratch_types=[
            pltpu.VMEM((BPW,), jnp.int32),
            pltpu.VMEM((BPW,), jnp.int32),
            pltpu.VMEM((BPW,), jnp.int32),
            pltpu.VMEM((6, BPW), jnp.int32),
            pltpu.VMEM((CH, W), jnp.float32),
            pltpu.VMEM((CH, W), jnp.float32),
            pltpu.VMEM((CH, W), jnp.float32),
            pltpu.VMEM((CH, W), jnp.float32),
            pltpu.VMEM((CH, W), jnp.float32),
            pltpu.VMEM((CH, W), jnp.float32),
            pltpu.VMEM((L,), jnp.float32),
            pltpu.VMEM((BPW,), jnp.float32),
            pltpu.VMEM((BPW,), jnp.float32),
            pltpu.SemaphoreType.DMA,
        ],
    )
    return f(u1, i1, j1, eu2, ei2, ub2, ib2, b16)


def kernel(u, i, j, c, embed_user, embed_item, u_bias, i_bias, bias_):
    del c
    u1 = u.astype(jnp.int32)
    i1 = i.astype(jnp.int32)
    j1 = j.astype(jnp.int32)
    eu2 = jnp.pad(embed_user, ((0, 0), (0, W - D)))
    ei2 = jnp.pad(embed_item, ((0, 0), (0, W - D)))
    nu = u_bias.shape[0]
    ni = i_bias.shape[0]
    ub2 = jnp.pad(u_bias.reshape(-1), (0, (-nu) % W)).reshape(-1, W)
    ib2 = jnp.pad(i_bias.reshape(-1), (0, (-ni) % W)).reshape(-1, W)
    b16 = jnp.broadcast_to(bias_, (L,))
    return tuple(_pair_fm(u1, i1, j1, eu2, ei2, ub2, ib2, b16))


# speedup vs baseline: 5.0218x; 1.9979x over previous
import jax
import jax.numpy as jnp
from jax import lax
from jax.experimental import pallas as pl
from jax.experimental.pallas import tpu as pltpu
from jax.experimental.pallas import tpu_sc as plsc

B = 16384
D = 64
R = 8
NC = 2
NS = 16
NW = NC * NS
BPW = B // NW
L = 16
NG = BPW // L


def _load_gather(ref, indices):
    return plsc.load_gather(ref, indices)


def _fm_body(u_hbm, i_hbm, j_hbm, eu_hbm, ei_hbm,
             out_i, out_j,
             uidx, iidx, jidx, ub0, ib0, jb0, ub1, ib1, jb1,
             res_i, res_j, sem0, sem1):
    wid = lax.axis_index("c") * NS + lax.axis_index("s")
    base = wid * BPW

    pltpu.sync_copy(u_hbm.at[pl.ds(base, BPW)], uidx)
    pltpu.sync_copy(i_hbm.at[pl.ds(base, BPW)], iidx)
    pltpu.sync_copy(j_hbm.at[pl.ds(base, BPW)], jidx)

    iota16 = lax.iota(jnp.int32, L)
    slots = ((ub0, ib0, jb0, sem0), (ub1, ib1, jb1, sem1))

    def fire(g, slot):
        ub, ib, jb, sem = slot
        s = pl.ds(g * L, L)
        utv = uidx[s] >> 3
        itv = iidx[s] >> 3
        jtv = jidx[s] >> 3
        for l in range(L):
            pltpu.async_copy(eu_hbm.at[utv[l]], ub.at[l], sem)
            pltpu.async_copy(ei_hbm.at[itv[l]], ib.at[l], sem)
            pltpu.async_copy(ei_hbm.at[jtv[l]], jb.at[l], sem)

    def drain(slot):
        ub, ib, jb, sem = slot
        for l in range(L):
            pltpu.make_async_copy(eu_hbm.at[0], ub.at[l], sem).wait()
            pltpu.make_async_copy(ei_hbm.at[0], ib.at[l], sem).wait()
            pltpu.make_async_copy(ei_hbm.at[0], jb.at[l], sem).wait()

    def compute(g, slot):
        ub, ib, jb, _ = slot
        s = pl.ds(g * L, L)
        us = uidx[s] & 7
        isb = iidx[s] & 7
        jsb = jidx[s] & 7

        def fbody(f, carry):
            acc_i, acc_j = carry
            fv = iota16 * 0 + f
            ue = _load_gather(ub, [iota16, us, fv])
            ie = _load_gather(ib, [iota16, isb, fv])
            je = _load_gather(jb, [iota16, jsb, fv])
            return acc_i + ue * ie, acc_j + ue * je

        acc0 = jnp.zeros((L,), jnp.float32)
        acc_i, acc_j = lax.fori_loop(0, D, fbody, (acc0, acc0), unroll=8)
        res_i[s] = acc_i
        res_j[s] = acc_j

    fire(0, slots[0])

    def body(g2, _):
        g = 2 * g2
        fire(g + 1, slots[1])
        drain(slots[0])
        compute(g, slots[0])

        @pl.when(g + 2 < NG)
        def _():
            fire(g + 2, slots[0])

        drain(slots[1])
        compute(g + 1, slots[1])
        return 0

    lax.fori_loop(0, NG // 2, body, 0)

    pltpu.sync_copy(res_i, out_i.at[pl.ds(base, BPW)])
    pltpu.sync_copy(res_j, out_j.at[pl.ds(base, BPW)])


@jax.jit
def _pair_fm(u1, i1, j1, eu3, ei3):
    mesh = plsc.VectorSubcoreMesh(core_axis_name="c", subcore_axis_name="s",
                                  num_cores=NC, num_subcores=NS)
    f = pl.kernel(
        _fm_body,
        out_type=[jax.ShapeDtypeStruct((B,), jnp.float32),
                  jax.ShapeDtypeStruct((B,), jnp.float32)],
        mesh=mesh,
        compiler_params=pltpu.CompilerParams(needs_layout_passes=False,
                                             use_tc_tiling_on_sc=True),
        scratch_types=[
            pltpu.VMEM((BPW,), jnp.int32),
            pltpu.VMEM((BPW,), jnp.int32),
            pltpu.VMEM((BPW,), jnp.int32),
            pltpu.VMEM((L, R, D), jnp.float32),
            pltpu.VMEM((L, R, D), jnp.float32),
            pltpu.VMEM((L, R, D), jnp.float32),
            pltpu.VMEM((L, R, D), jnp.float32),
            pltpu.VMEM((L, R, D), jnp.float32),
            pltpu.VMEM((L, R, D), jnp.float32),
            pltpu.VMEM((BPW,), jnp.float32),
            pltpu.VMEM((BPW,), jnp.float32),
            pltpu.SemaphoreType.DMA,
            pltpu.SemaphoreType.DMA,
        ],
    )
    return f(u1, i1, j1, eu3, ei3)


def kernel(u, i, j, c, embed_user, embed_item, u_bias, i_bias, bias_):
    del c, u_bias, i_bias, bias_
    u1 = u.astype(jnp.int32)
    i1 = i.astype(jnp.int32)
    j1 = j.astype(jnp.int32)
    eu3 = embed_user.reshape(-1, R, D)
    ei3 = embed_item.reshape(-1, R, D)
    return tuple(_pair_fm(u1, i1, j1, eu3, ei3))
